# baseline (speedup 1.0000x reference)
import jax
import jax.numpy as jnp
from jax import lax
from jax.experimental import pallas as pl
from jax.experimental.pallas import tpu as pltpu

_NUM_EXPERTS = 64
_TOP_K = 8
_TOKEN_BLOCK = 1024


def _moe_gate_body(x_ref, w_ref, probs_ref, idx_ref, wts_ref, counts_ref,
                   wt_ref):
    @pl.when(pl.program_id(0) == 0)
    def _transpose_w():
        wt_ref[...] = jnp.transpose(w_ref[...])

    x = x_ref[...]
    logits = jnp.dot(x, wt_ref[...],
                     preferred_element_type=jnp.float32)

    m = jnp.max(logits, axis=-1, keepdims=True)
    e = jnp.exp(logits - m)
    denom = jnp.sum(e, axis=-1, keepdims=True)
    probs = e / denom
    probs_ref[...] = probs

    tb, n_exp = probs.shape
    work = jnp.transpose(probs)
    row_f = lax.broadcasted_iota(jnp.int32, (n_exp, tb), 0).astype(
        jnp.float32)
    big = jnp.float32(n_exp)
    idx_rows = []
    val_rows = []
    for _ in range(_TOP_K):
        mx = jnp.max(work, axis=0, keepdims=True)
        sel = jnp.min(jnp.where(work == mx, row_f, big), axis=0,
                      keepdims=True)
        idx_rows.append(sel)
        val_rows.append(mx)
        work = jnp.where(row_f == sel, -1.0, work)

    idx_t = jnp.concatenate(idx_rows, axis=0)
    val_t = jnp.concatenate(val_rows, axis=0)
    wts_t = val_t / jnp.sum(val_t, axis=0, keepdims=True)
    idx_ref[...] = jnp.transpose(idx_t).astype(jnp.int32)
    wts_ref[...] = jnp.transpose(wts_t)

    selected = jnp.where(work < 0.0, 1.0, 0.0)
    counts = jnp.sum(selected, axis=1).reshape(1, n_exp)

    @pl.when(pl.program_id(0) == 0)
    def _init():
        counts_ref[...] = jnp.zeros_like(counts_ref)

    counts_ref[...] += counts


def kernel(hidden_states, W):
    b, s, h = hidden_states.shape
    n_exp, _ = W.shape
    tokens = b * s
    tb = _TOKEN_BLOCK
    x = hidden_states.reshape(tokens, h)

    probs, idx, wts, counts = pl.pallas_call(
        _moe_gate_body,
        grid=(tokens // tb,),
        in_specs=[
            pl.BlockSpec((tb, h), lambda i: (i, 0)),
            pl.BlockSpec((n_exp, h), lambda i: (0, 0)),
        ],
        scratch_shapes=[pltpu.VMEM((h, n_exp), jnp.float32)],
        out_specs=[
            pl.BlockSpec((tb, n_exp), lambda i: (i, 0)),
            pl.BlockSpec((tb, _TOP_K), lambda i: (i, 0)),
            pl.BlockSpec((tb, _TOP_K), lambda i: (i, 0)),
            pl.BlockSpec((1, n_exp), lambda i: (0, 0)),
        ],
        out_shape=[
            jax.ShapeDtypeStruct((tokens, n_exp), jnp.float32),
            jax.ShapeDtypeStruct((tokens, _TOP_K), jnp.int32),
            jax.ShapeDtypeStruct((tokens, _TOP_K), jnp.float32),
            jax.ShapeDtypeStruct((1, n_exp), jnp.float32),
        ],
    )(x, W)

    expert_indices = idx.reshape(b, s, _TOP_K)
    routing_weights = wts.reshape(b, s, _TOP_K)
    expert_counts = counts.reshape(n_exp).astype(jnp.int64)
    router_probs = probs.reshape(b, s, n_exp)
    return (expert_indices, routing_weights, expert_counts, router_probs)
